# baseline (speedup 1.0000x reference)
import jax
import jax.numpy as jnp
from jax import lax
from jax.experimental import pallas as pl
from jax.experimental.pallas import tpu as pltpu
from jax.experimental.pallas import tpu_sc as plsc

BATCH = 4
SEQ_LEN = 8192
D_MODEL = 768

BS = 2048
_SC_ROWS = 2048
_S0 = SEQ_LEN - _SC_ROWS


_SC_NC = 2
_SC_NS = 16
_SC_NW = _SC_NC * _SC_NS
_SC_RPW = _SC_ROWS // _SC_NW
_SC_CH = 32
_SC_NCHUNK = _SC_RPW // _SC_CH


def _sc_body(x_hbm, pos_hbm, out_hbm, x_v0, x_v1, pos_v,
             li0, li1, lo0, lo1):
    wid = lax.axis_index("s") * _SC_NC + lax.axis_index("c")
    seq0 = _S0 + wid * _SC_RPW
    out0 = wid * _SC_RPW
    bufs = (x_v0, x_v1)
    lsems = (li0, li1)
    ssems = (lo0, lo1)

    nsteps = _SC_NCHUNK
    loads = [None] * nsteps
    stores = [None] * nsteps

    def start_load(t):
        loads[t] = pltpu.async_copy(
            x_hbm.at[BATCH - 1, pl.ds(seq0 + t * _SC_CH, _SC_CH)],
            bufs[t % 2], lsems[t % 2])

    start_load(0)
    for t in range(nsteps):
        cur = bufs[t % 2]
        if t + 1 < nsteps:
            if t - 1 >= 0:
                stores[t - 1].wait()
            start_load(t + 1)
        pltpu.sync_copy(pos_hbm.at[pl.ds(seq0 + t * _SC_CH, _SC_CH)], pos_v)
        loads[t].wait()

        @plsc.parallel_loop(0, _SC_CH, step=1, unroll=2)
        def _add(r):
            for c in range(D_MODEL // 16):
                sl = pl.ds(c * 16, 16)
                cur[r, sl] = cur[r, sl] + pos_v[r, sl]

        stores[t] = pltpu.async_copy(
            cur, out_hbm.at[pl.ds(out0 + t * _SC_CH, _SC_CH)], ssems[t % 2])
    if nsteps >= 2:
        stores[nsteps - 2].wait()
    stores[nsteps - 1].wait()


def _sc_tail(x, pos_table):
    mesh = plsc.VectorSubcoreMesh(core_axis_name="c", subcore_axis_name="s")
    run = pl.kernel(
        _sc_body,
        out_type=jax.ShapeDtypeStruct((_SC_ROWS, D_MODEL), jnp.float32),
        mesh=mesh,
        scratch_types=[
            pltpu.VMEM((_SC_CH, D_MODEL), jnp.float32),
            pltpu.VMEM((_SC_CH, D_MODEL), jnp.float32),
            pltpu.VMEM((_SC_CH, D_MODEL), jnp.float32),
            pltpu.SemaphoreType.DMA,
            pltpu.SemaphoreType.DMA,
            pltpu.SemaphoreType.DMA,
            pltpu.SemaphoreType.DMA,
        ],
    )
    return run(x, pos_table)


_N_SEQ = SEQ_LEN // BS
_N_HEAD = _S0 // BS


def _tc_xb(s, b):
    tail = (s == _N_SEQ - 1) & (b == BATCH - 1)
    return jnp.where(tail, BATCH - 2, b)


def _tc_main_body(x_ref, pos_ref, out_ref):
    out_ref[...] = x_ref[...] + pos_ref[...][None]


def _tc_main(x, pos_table):
    return pl.pallas_call(
        _tc_main_body,
        grid=(_N_SEQ, BATCH),
        in_specs=[
            pl.BlockSpec((1, BS, D_MODEL), lambda s, b: (_tc_xb(s, b), s, 0)),
            pl.BlockSpec((BS, D_MODEL), lambda s, b: (s, 0)),
        ],
        out_specs=pl.BlockSpec((1, BS, D_MODEL),
                               lambda s, b: (_tc_xb(s, b), s, 0)),
        out_shape=jax.ShapeDtypeStruct((BATCH, SEQ_LEN, D_MODEL), jnp.float32),
    )(x, pos_table)


def _patch_body(main_ref, sc_ref, out_ref):
    out_ref[...] = sc_ref[...][None]


def _patch(main, sc_out):
    return pl.pallas_call(
        _patch_body,
        grid=(_SC_ROWS // BS,),
        in_specs=[
            pl.BlockSpec(memory_space=pl.ANY),
            pl.BlockSpec((BS, D_MODEL), lambda s: (s, 0)),
        ],
        out_specs=pl.BlockSpec((1, BS, D_MODEL),
                               lambda s: (BATCH - 1, _N_HEAD + s, 0)),
        out_shape=jax.ShapeDtypeStruct((BATCH, SEQ_LEN, D_MODEL), jnp.float32),
        input_output_aliases={0: 0},
    )(main, sc_out)


def kernel(x, pos_table):
    sc_out = _sc_tail(x, pos_table)
    main = _tc_main(x, pos_table)
    return _patch(main, sc_out)
